# baseline (speedup 1.0000x reference)
import functools

import jax
import jax.numpy as jnp
from jax import lax
from jax.experimental import pallas as pl
from jax.experimental.pallas import tpu as pltpu
from jax.experimental.pallas import tpu_sc as plsc

N_NODES = 10000
N_EDGES = 320000
NFEAT = 128
NHID = 128
NCLASS = 64

E_PAD1 = 331776
E_PAD2 = 327680
C = 128
ROWS_PER_TILE = N_NODES // 16
_ZCHUNKS = [(0, 128), (128, 128), (256, 128), (384, 128), (512, 113)]


def _mm_body(x_ref, w_ref, o_ref):
    o_ref[...] = jnp.dot(x_ref[...], w_ref[...],
                         preferred_element_type=jnp.float32)


def _matmul(x, w, bm=1000):
    m, k = x.shape
    n = w.shape[1]
    return pl.pallas_call(
        _mm_body,
        grid=(m // bm,),
        in_specs=[pl.BlockSpec((bm, k), lambda i: (i, 0)),
                  pl.BlockSpec((k, n), lambda i: (0, 0))],
        out_specs=pl.BlockSpec((bm, n), lambda i: (i, 0)),
        out_shape=jax.ShapeDtypeStruct((m, n), jnp.float32),
    )(x, w)


def _mm2_body(p_ref, w_ref, o_ref):
    h = jnp.maximum(p_ref[0] + p_ref[1], 0.0)
    o_ref[...] = jnp.dot(h, w_ref[...], preferred_element_type=jnp.float32)


def _relu_sum_matmul(p, w, bm=1000):
    _, m, k = p.shape
    n = w.shape[1]
    return pl.pallas_call(
        _mm2_body,
        grid=(m // bm,),
        in_specs=[pl.BlockSpec((2, bm, k), lambda i: (0, i, 0)),
                  pl.BlockSpec((k, n), lambda i: (0, 0))],
        out_specs=pl.BlockSpec((bm, n), lambda i: (i, 0)),
        out_shape=jax.ShapeDtypeStruct((m, n), jnp.float32),
    )(p, w)


def _sum2_body(p_ref, o_ref):
    o_ref[...] = p_ref[0] + p_ref[1]


def _sum2(p, bm=1000):
    _, m, n = p.shape
    return pl.pallas_call(
        _sum2_body,
        grid=(m // bm,),
        in_specs=[pl.BlockSpec((2, bm, n), lambda i: (0, i, 0))],
        out_specs=pl.BlockSpec((bm, n), lambda i: (i, 0)),
        out_shape=jax.ShapeDtypeStruct((m, n), jnp.float32),
    )(p)


_GATHER_DNUMS = lax.GatherDimensionNumbers(
    offset_dims=(), collapsed_slice_dims=(0,), start_index_map=(0,))


def _lane_splat(vec, lane):
    idx = jnp.full((16, 1), lane, jnp.int32)
    return lax.gather(vec, idx, _GATHER_DNUMS, slice_sizes=(1,),
                      mode=lax.GatherScatterMode.PROMISE_IN_BOUNDS)


def _make_agg(F, R, LA, e_pad):
    nslab = 32
    ept = e_pad // nslab
    nch = ept // C
    assert nch % R == 0 and LA < R
    DW = R - LA
    mesh = plsc.VectorSubcoreMesh(core_axis_name="c", subcore_axis_name="s")

    @functools.partial(
        pl.kernel,
        out_type=jax.ShapeDtypeStruct((2, N_NODES, F), jnp.float32),
        mesh=mesh,
        compiler_params=pltpu.CompilerParams(use_tc_tiling_on_sc=False,
                                             needs_layout_passes=False),
        scratch_types=(
            [pltpu.VMEM_SHARED((N_NODES, F), jnp.float32),
             pltpu.VMEM((R, 3, C), jnp.int32),
             pltpu.VMEM((R, C), jnp.int32)]
            + [pltpu.VMEM((C, F), jnp.float32) for _ in range(R)]
            + [pltpu.SemaphoreType.DMA] * (3 * R)
        ),
    )
    def agg(h_hbm, edges_hbm, out_hbm, acc_sh, stage_v, dstc_v, *rest):
        rows, sems = rest[:R], rest[R:]
        ssems, gsems, scsems = sems[0:R], sems[R:2 * R], sems[2 * R:3 * R]
        c = lax.axis_index("c")
        s = lax.axis_index("s")
        wid = s * 2 + c

        def stage_start(k, b):
            pltpu.async_copy(edges_hbm.at[wid, k], stage_v.at[b], ssems[b])

        def stage_wait(k, b):
            pltpu.make_async_copy(edges_hbm.at[wid, k], stage_v.at[b],
                                  ssems[b]).wait()

        def gather_start(b):
            pltpu.async_copy(h_hbm.at[stage_v.at[b, 0]], rows[b], gsems[b])

        def gather_wait(b):
            pltpu.make_async_copy(h_hbm.at[stage_v.at[b, 0]], rows[b],
                                  gsems[b]).wait()

        def scatter_start(b):
            pltpu.async_copy(rows[b], acc_sh.at[dstc_v.at[b]], scsems[b],
                             add=True)

        def scatter_wait(b):
            pltpu.make_async_copy(rows[b], acc_sh.at[dstc_v.at[b]],
                                  scsems[b]).wait()

        for b in range(R):
            stage_start(b, b)

        def zrow(r, carry):
            for j in range(F // 16):
                rows[0][r, pl.ds(j * 16, 16)] = jnp.zeros((16,), jnp.float32)
            return carry
        lax.fori_loop(0, C, zrow, 0)
        for (z0, zn) in _ZCHUNKS:
            pltpu.sync_copy(rows[0].at[pl.ds(0, zn)],
                            acc_sh.at[pl.ds(s * ROWS_PER_TILE + z0, zn)])
        plsc.subcore_barrier()

        for b in range(LA):
            stage_wait(b, b)
            gather_start(b)

        def step(kk, carry):
            for b in range(R):
                k = kk * R + b
                gather_wait(b)

                def group(g, carry2):
                    wv = plsc.bitcast(stage_v[b, 2, pl.ds(g * 16, 16)],
                                      jnp.float32)
                    for l in range(16):
                        splat = _lane_splat(wv, l)
                        e = g * 16 + l
                        for j in range(F // 16):
                            rows[b][e, pl.ds(j * 16, 16)] = (
                                rows[b][e, pl.ds(j * 16, 16)] * splat)
                    return carry2
                lax.fori_loop(0, C // 16, group, 0)

                for j in range(C // 16):
                    dstc_v[b, pl.ds(j * 16, 16)] = stage_v[b, 1,
                                                           pl.ds(j * 16, 16)]
                scatter_start(b)

                @pl.when(k + R < nch)
                def _():
                    stage_start(k + R, b)

                if b < DW:
                    @pl.when(kk >= 1)
                    def _():
                        scatter_wait((b - DW) % R)
                else:
                    scatter_wait((b - DW) % R)

                @pl.when(k + LA < nch)
                def _():
                    stage_wait(k + LA, (b + LA) % R)
                    gather_start((b + LA) % R)
            return carry
        lax.fori_loop(0, nch // R, step, 0)
        for d in range(DW):
            scatter_wait((nch - DW + d) % R)

        plsc.subcore_barrier()
        for (z0, zn) in _ZCHUNKS:
            r0 = s * ROWS_PER_TILE + z0
            pltpu.sync_copy(acc_sh.at[pl.ds(r0, zn)],
                            out_hbm.at[c, pl.ds(r0, zn)])

    return agg


_agg_l1 = _make_agg(NHID, R=3, LA=2, e_pad=E_PAD1)
_agg_l2 = _make_agg(NCLASS, R=4, LA=2, e_pad=E_PAD2)


def _pad_edges(ei, ew, e_pad):
    nslab = 32
    npad = e_pad - N_EDGES
    nch = e_pad // nslab // C
    spread = jnp.arange(npad, dtype=jnp.int32) % N_NODES
    src = jnp.concatenate([ei[0], spread]).reshape(nslab, nch, 1, C)
    dst = jnp.concatenate([ei[1], spread]).reshape(nslab, nch, 1, C)
    w = lax.bitcast_convert_type(
        jnp.pad(ew, (0, npad)), jnp.int32).reshape(nslab, nch, 1, C)
    return jnp.concatenate([src, dst, w], axis=2)


def kernel(x, edge_index1, edge_index2, edge_weight1, edge_weight2, W1, W2):
    e1 = _pad_edges(edge_index1, edge_weight1, E_PAD1)
    e2 = _pad_edges(edge_index2, edge_weight2, E_PAD2)

    h1 = _matmul(x, W1)
    p1 = _agg_l1(h1, e1)
    h2 = _relu_sum_matmul(p1, W2)
    p2 = _agg_l2(h2, e2)
    return _sum2(p2)


# speedup vs baseline: 1.0098x; 1.0098x over previous
import functools

import jax
import jax.numpy as jnp
from jax import lax
from jax.experimental import pallas as pl
from jax.experimental.pallas import tpu as pltpu
from jax.experimental.pallas import tpu_sc as plsc

N_NODES = 10000
N_EDGES = 320000
NFEAT = 128
NHID = 128
NCLASS = 64

E_PAD1 = 331776
E_PAD2 = 327680
C = 128
ROWS_PER_TILE = 624
_ZCHUNKS = [(0, 128), (128, 128), (256, 128), (384, 128), (512, 112)]
_ZTAIL = (ROWS_PER_TILE * 16, N_NODES - ROWS_PER_TILE * 16)


def _mm_body(x_ref, w_ref, o_ref):
    o_ref[...] = jnp.dot(x_ref[...], w_ref[...],
                         preferred_element_type=jnp.float32)


def _matmul(x, w, bm=1000):
    m, k = x.shape
    n = w.shape[1]
    return pl.pallas_call(
        _mm_body,
        grid=(m // bm,),
        in_specs=[pl.BlockSpec((bm, k), lambda i: (i, 0)),
                  pl.BlockSpec((k, n), lambda i: (0, 0))],
        out_specs=pl.BlockSpec((bm, n), lambda i: (i, 0)),
        out_shape=jax.ShapeDtypeStruct((m, n), jnp.float32),
    )(x, w)


def _mm2_body(p_ref, w_ref, o_ref):
    h = jnp.maximum(p_ref[0] + p_ref[1], 0.0)
    o_ref[...] = jnp.dot(h, w_ref[...], preferred_element_type=jnp.float32)


def _relu_sum_matmul(p, w, bm=1000):
    _, m, k = p.shape
    n = w.shape[1]
    return pl.pallas_call(
        _mm2_body,
        grid=(m // bm,),
        in_specs=[pl.BlockSpec((2, bm, k), lambda i: (0, i, 0)),
                  pl.BlockSpec((k, n), lambda i: (0, 0))],
        out_specs=pl.BlockSpec((bm, n), lambda i: (i, 0)),
        out_shape=jax.ShapeDtypeStruct((m, n), jnp.float32),
    )(p, w)


def _sum2_body(p_ref, o_ref):
    o_ref[...] = p_ref[0] + p_ref[1]


def _sum2(p, bm=1000):
    _, m, n = p.shape
    return pl.pallas_call(
        _sum2_body,
        grid=(m // bm,),
        in_specs=[pl.BlockSpec((2, bm, n), lambda i: (0, i, 0))],
        out_specs=pl.BlockSpec((bm, n), lambda i: (i, 0)),
        out_shape=jax.ShapeDtypeStruct((m, n), jnp.float32),
    )(p)


_GATHER_DNUMS = lax.GatherDimensionNumbers(
    offset_dims=(), collapsed_slice_dims=(0,), start_index_map=(0,))


def _lane_splat(vec, lane):
    idx = jnp.full((16, 1), lane, jnp.int32)
    return lax.gather(vec, idx, _GATHER_DNUMS, slice_sizes=(1,),
                      mode=lax.GatherScatterMode.PROMISE_IN_BOUNDS)


def _make_agg(F, R, LA, e_pad, tc_tiling=False):
    nslab = 32
    ept = e_pad // nslab
    nch = ept // C
    assert nch % R == 0 and LA < R
    DW = R - LA
    mesh = plsc.VectorSubcoreMesh(core_axis_name="c", subcore_axis_name="s")

    @functools.partial(
        pl.kernel,
        out_type=jax.ShapeDtypeStruct((2, N_NODES, F), jnp.float32),
        mesh=mesh,
        compiler_params=pltpu.CompilerParams(
            use_tc_tiling_on_sc=(None if tc_tiling else False),
            needs_layout_passes=False),
        scratch_types=(
            [pltpu.VMEM_SHARED((N_NODES, F), jnp.float32),
             pltpu.VMEM((R, C), jnp.int32),
             pltpu.VMEM((R, C), jnp.int32),
             pltpu.VMEM((R, C), jnp.float32),
             pltpu.VMEM((R, C), jnp.int32)]
            + [pltpu.VMEM((C, F), jnp.float32) for _ in range(R)]
            + [pltpu.SemaphoreType.DMA] * (3 * R)
        ),
    )
    def agg(h_hbm, src_hbm, dst_hbm, w_hbm, out_hbm,
            acc_sh, ssrc_v, sdst_v, sw_v, dstc_v, *rest):
        rows, sems = rest[:R], rest[R:]
        ssems, gsems, scsems = sems[0:R], sems[R:2 * R], sems[2 * R:3 * R]
        c = lax.axis_index("c")
        s = lax.axis_index("s")
        wid = s * 2 + c

        def stage_start(k, b):
            pltpu.async_copy(src_hbm.at[wid, k], ssrc_v.at[b], ssems[b])
            pltpu.async_copy(dst_hbm.at[wid, k], sdst_v.at[b], ssems[b])
            pltpu.async_copy(w_hbm.at[wid, k], sw_v.at[b], ssems[b])

        def stage_wait(k, b):
            pltpu.make_async_copy(src_hbm.at[wid, k], ssrc_v.at[b],
                                  ssems[b]).wait()
            pltpu.make_async_copy(dst_hbm.at[wid, k], sdst_v.at[b],
                                  ssems[b]).wait()
            pltpu.make_async_copy(w_hbm.at[wid, k], sw_v.at[b],
                                  ssems[b]).wait()

        def gather_start(b):
            pltpu.async_copy(h_hbm.at[ssrc_v.at[b]], rows[b], gsems[b])

        def gather_wait(b):
            pltpu.make_async_copy(h_hbm.at[ssrc_v.at[b]], rows[b],
                                  gsems[b]).wait()

        def scatter_start(b):
            pltpu.async_copy(rows[b], acc_sh.at[dstc_v.at[b]], scsems[b],
                             add=True)

        def scatter_wait(b):
            pltpu.make_async_copy(rows[b], acc_sh.at[dstc_v.at[b]],
                                  scsems[b]).wait()

        for b in range(R):
            stage_start(b, b)

        def zrow(r, carry):
            for j in range(F // 16):
                rows[0][r, pl.ds(j * 16, 16)] = jnp.zeros((16,), jnp.float32)
            return carry
        lax.fori_loop(0, C, zrow, 0)
        for (z0, zn) in _ZCHUNKS:
            pltpu.sync_copy(rows[0].at[pl.ds(0, zn)],
                            acc_sh.at[pl.ds(s * ROWS_PER_TILE + z0, zn)])

        @pl.when(s == 15)
        def _():
            pltpu.sync_copy(rows[0].at[pl.ds(0, _ZTAIL[1])],
                            acc_sh.at[pl.ds(_ZTAIL[0], _ZTAIL[1])])
        plsc.subcore_barrier()

        for b in range(LA):
            stage_wait(b, b)
            gather_start(b)

        def step(kk, carry):
            for b in range(R):
                k = kk * R + b
                gather_wait(b)

                def group(g, carry2):
                    wv = sw_v[b, pl.ds(g * 16, 16)]
                    for l in range(16):
                        splat = _lane_splat(wv, l)
                        e = g * 16 + l
                        for j in range(F // 16):
                            rows[b][e, pl.ds(j * 16, 16)] = (
                                rows[b][e, pl.ds(j * 16, 16)] * splat)
                    return carry2
                lax.fori_loop(0, C // 16, group, 0)

                for j in range(C // 16):
                    dstc_v[b, pl.ds(j * 16, 16)] = sdst_v[b, pl.ds(j * 16, 16)]
                scatter_start(b)

                @pl.when(k + R < nch)
                def _():
                    stage_start(k + R, b)

                if b < DW:
                    @pl.when(kk >= 1)
                    def _():
                        scatter_wait((b - DW) % R)
                else:
                    scatter_wait((b - DW) % R)

                @pl.when(k + LA < nch)
                def _():
                    stage_wait(k + LA, (b + LA) % R)
                    gather_start((b + LA) % R)
            return carry
        lax.fori_loop(0, nch // R, step, 0)
        for d in range(DW):
            scatter_wait((nch - DW + d) % R)

        plsc.subcore_barrier()
        for (z0, zn) in _ZCHUNKS:
            r0 = s * ROWS_PER_TILE + z0
            pltpu.sync_copy(acc_sh.at[pl.ds(r0, zn)],
                            out_hbm.at[c, pl.ds(r0, zn)])

        @pl.when(s == 15)
        def _():
            pltpu.sync_copy(acc_sh.at[pl.ds(_ZTAIL[0], _ZTAIL[1])],
                            out_hbm.at[c, pl.ds(_ZTAIL[0], _ZTAIL[1])])

    return agg


_agg_l1 = _make_agg(NHID, R=3, LA=2, e_pad=E_PAD1)
_agg_l2 = _make_agg(NCLASS, R=4, LA=2, e_pad=E_PAD2)


def _pad_edges(ei, ew, e_pad):
    nslab = 32
    npad = e_pad - N_EDGES
    nch = e_pad // nslab // C
    spread = jnp.arange(npad, dtype=jnp.int32) % N_NODES
    src = jnp.concatenate([ei[0], spread]).reshape(nslab, nch, C)
    dst = jnp.concatenate([ei[1], spread]).reshape(nslab, nch, C)
    w = jnp.pad(ew, (0, npad)).reshape(nslab, nch, C)
    return src, dst, w


def kernel(x, edge_index1, edge_index2, edge_weight1, edge_weight2, W1, W2):
    s1, d1, w1 = _pad_edges(edge_index1, edge_weight1, E_PAD1)
    s2, d2, w2 = _pad_edges(edge_index2, edge_weight2, E_PAD2)

    h1 = _matmul(x, W1)
    p1 = _agg_l1(h1, s1, d1, w1)
    h2 = _relu_sum_matmul(p1, W2)
    p2 = _agg_l2(h2, s2, d2, w2)
    return _sum2(p2)


# speedup vs baseline: 1.2143x; 1.2025x over previous
import functools

import jax
import jax.numpy as jnp
from jax import lax
from jax.experimental import pallas as pl
from jax.experimental.pallas import tpu as pltpu
from jax.experimental.pallas import tpu_sc as plsc

N_NODES = 10000
N_EDGES = 320000
NFEAT = 128
NHID = 128
NCLASS = 64

E_PAD1 = 331776
E_PAD2 = 327680
C = 128
ROWS_PER_TILE = 624
_ZCHUNKS = [(0, 128), (128, 128), (256, 128), (384, 128), (512, 112)]
_ZTAIL = (ROWS_PER_TILE * 16, N_NODES - ROWS_PER_TILE * 16)


def _mm_body(x_ref, w_ref, o_ref):
    o_ref[...] = jnp.dot(x_ref[...], w_ref[...],
                         preferred_element_type=jnp.float32)


def _matmul(x, w, bm=1000):
    m, k = x.shape
    n = w.shape[1]
    return pl.pallas_call(
        _mm_body,
        grid=(m // bm,),
        in_specs=[pl.BlockSpec((bm, k), lambda i: (i, 0)),
                  pl.BlockSpec((k, n), lambda i: (0, 0))],
        out_specs=pl.BlockSpec((bm, n), lambda i: (i, 0)),
        out_shape=jax.ShapeDtypeStruct((m, n), jnp.float32),
    )(x, w)


def _mm2_body(p_ref, w_ref, o_ref):
    h = jnp.maximum(p_ref[0] + p_ref[1], 0.0)
    o_ref[...] = jnp.dot(h, w_ref[...], preferred_element_type=jnp.float32)


def _relu_sum_matmul(p, w, bm=1000):
    _, m, k = p.shape
    n = w.shape[1]
    return pl.pallas_call(
        _mm2_body,
        grid=(m // bm,),
        in_specs=[pl.BlockSpec((2, bm, k), lambda i: (0, i, 0)),
                  pl.BlockSpec((k, n), lambda i: (0, 0))],
        out_specs=pl.BlockSpec((bm, n), lambda i: (i, 0)),
        out_shape=jax.ShapeDtypeStruct((m, n), jnp.float32),
    )(p, w)


def _sum2_body(p_ref, o_ref):
    o_ref[...] = p_ref[0, :, :NCLASS] + p_ref[1, :, :NCLASS]


def _sum2(p, bm=1000):
    _, m, n = p.shape
    return pl.pallas_call(
        _sum2_body,
        grid=(m // bm,),
        in_specs=[pl.BlockSpec((2, bm, n), lambda i: (0, i, 0))],
        out_specs=pl.BlockSpec((bm, NCLASS), lambda i: (i, 0)),
        out_shape=jax.ShapeDtypeStruct((m, NCLASS), jnp.float32),
    )(p)


_GATHER_DNUMS = lax.GatherDimensionNumbers(
    offset_dims=(), collapsed_slice_dims=(0,), start_index_map=(0,))


def _lane_splat(vec, lane):
    idx = jnp.full((16, 1), lane, jnp.int32)
    return lax.gather(vec, idx, _GATHER_DNUMS, slice_sizes=(1,),
                      mode=lax.GatherScatterMode.PROMISE_IN_BOUNDS)


def _make_agg(F, R, LA, e_pad, tc_tiling=False):
    nslab = 32
    ept = e_pad // nslab
    nch = ept // C
    assert nch % R == 0 and LA < R
    DW = R - LA
    mesh = plsc.VectorSubcoreMesh(core_axis_name="c", subcore_axis_name="s")

    @functools.partial(
        pl.kernel,
        out_type=jax.ShapeDtypeStruct((2, N_NODES, F), jnp.float32),
        mesh=mesh,
        compiler_params=pltpu.CompilerParams(
            use_tc_tiling_on_sc=(None if tc_tiling else False),
            needs_layout_passes=False),
        scratch_types=(
            [pltpu.VMEM_SHARED((N_NODES, F), jnp.float32),
             pltpu.VMEM((R, C), jnp.int32),
             pltpu.VMEM((R, C), jnp.int32),
             pltpu.VMEM((R, C), jnp.float32),
             pltpu.VMEM((R, C), jnp.int32)]
            + [pltpu.VMEM((C, F), jnp.float32) for _ in range(R)]
            + [pltpu.SemaphoreType.DMA] * (3 * R)
        ),
    )
    def agg(h_hbm, src_hbm, dst_hbm, w_hbm, out_hbm,
            acc_sh, ssrc_v, sdst_v, sw_v, dstc_v, *rest):
        rows, sems = rest[:R], rest[R:]
        ssems, gsems, scsems = sems[0:R], sems[R:2 * R], sems[2 * R:3 * R]
        c = lax.axis_index("c")
        s = lax.axis_index("s")
        wid = s * 2 + c

        def stage_start(k, b):
            pltpu.async_copy(src_hbm.at[wid, k], ssrc_v.at[b], ssems[b])
            pltpu.async_copy(dst_hbm.at[wid, k], sdst_v.at[b], ssems[b])
            pltpu.async_copy(w_hbm.at[wid, k], sw_v.at[b], ssems[b])

        def stage_wait(k, b):
            pltpu.make_async_copy(src_hbm.at[wid, k], ssrc_v.at[b],
                                  ssems[b]).wait()
            pltpu.make_async_copy(dst_hbm.at[wid, k], sdst_v.at[b],
                                  ssems[b]).wait()
            pltpu.make_async_copy(w_hbm.at[wid, k], sw_v.at[b],
                                  ssems[b]).wait()

        def gather_start(b):
            pltpu.async_copy(h_hbm.at[ssrc_v.at[b]], rows[b], gsems[b])

        def gather_wait(b):
            pltpu.make_async_copy(h_hbm.at[ssrc_v.at[b]], rows[b],
                                  gsems[b]).wait()

        def scatter_start(b):
            pltpu.async_copy(rows[b], acc_sh.at[dstc_v.at[b]], scsems[b],
                             add=True)

        def scatter_wait(b):
            pltpu.make_async_copy(rows[b], acc_sh.at[dstc_v.at[b]],
                                  scsems[b]).wait()

        for b in range(R):
            stage_start(b, b)

        def zrow(r, carry):
            for j in range(F // 16):
                rows[0][r, pl.ds(j * 16, 16)] = jnp.zeros((16,), jnp.float32)
            return carry
        lax.fori_loop(0, C, zrow, 0)
        for (z0, zn) in _ZCHUNKS:
            pltpu.sync_copy(rows[0].at[pl.ds(0, zn)],
                            acc_sh.at[pl.ds(s * ROWS_PER_TILE + z0, zn)])

        @pl.when(s == 15)
        def _():
            pltpu.sync_copy(rows[0].at[pl.ds(0, _ZTAIL[1])],
                            acc_sh.at[pl.ds(_ZTAIL[0], _ZTAIL[1])])
        plsc.subcore_barrier()

        for b in range(LA):
            stage_wait(b, b)
            gather_start(b)

        def step(kk, carry):
            for b in range(R):
                k = kk * R + b
                gather_wait(b)

                def group(g, carry2):
                    wv = sw_v[b, pl.ds(g * 16, 16)]
                    for l in range(16):
                        splat = _lane_splat(wv, l)
                        e = g * 16 + l
                        for j in range(F // 16):
                            rows[b][e, pl.ds(j * 16, 16)] = (
                                rows[b][e, pl.ds(j * 16, 16)] * splat)
                    return carry2
                lax.fori_loop(0, C // 16, group, 0)

                for j in range(C // 16):
                    dstc_v[b, pl.ds(j * 16, 16)] = sdst_v[b, pl.ds(j * 16, 16)]
                scatter_start(b)

                @pl.when(k + R < nch)
                def _():
                    stage_start(k + R, b)

                if b < DW:
                    @pl.when(kk >= 1)
                    def _():
                        scatter_wait((b - DW) % R)
                else:
                    scatter_wait((b - DW) % R)

                @pl.when(k + LA < nch)
                def _():
                    stage_wait(k + LA, (b + LA) % R)
                    gather_start((b + LA) % R)
            return carry
        lax.fori_loop(0, nch // R, step, 0)
        for d in range(DW):
            scatter_wait((nch - DW + d) % R)

        plsc.subcore_barrier()
        for (z0, zn) in _ZCHUNKS:
            r0 = s * ROWS_PER_TILE + z0
            pltpu.sync_copy(acc_sh.at[pl.ds(r0, zn)],
                            out_hbm.at[c, pl.ds(r0, zn)])

        @pl.when(s == 15)
        def _():
            pltpu.sync_copy(acc_sh.at[pl.ds(_ZTAIL[0], _ZTAIL[1])],
                            out_hbm.at[c, pl.ds(_ZTAIL[0], _ZTAIL[1])])

    return agg


_agg_l1 = _make_agg(NHID, R=3, LA=2, e_pad=E_PAD1)
_agg_l2 = _agg_l1


def _pad_edges(ei, ew, e_pad):
    nslab = 32
    npad = e_pad - N_EDGES
    nch = e_pad // nslab // C
    spread = jnp.arange(npad, dtype=jnp.int32) % N_NODES
    src = jnp.concatenate([ei[0], spread]).reshape(nslab, nch, C)
    dst = jnp.concatenate([ei[1], spread]).reshape(nslab, nch, C)
    w = jnp.pad(ew, (0, npad)).reshape(nslab, nch, C)
    return src, dst, w


def kernel(x, edge_index1, edge_index2, edge_weight1, edge_weight2, W1, W2):
    s1, d1, w1 = _pad_edges(edge_index1, edge_weight1, E_PAD1)
    s2, d2, w2 = _pad_edges(edge_index2, edge_weight2, E_PAD1)
    w2p = jnp.pad(W2, ((0, 0), (0, NHID - NCLASS)))

    h1 = _matmul(x, W1)
    p1 = _agg_l1(h1, s1, d1, w1)
    h2 = _relu_sum_matmul(p1, w2p)
    p2 = _agg_l2(h2, s2, d2, w2)
    return _sum2(p2)
